# Initial kernel scaffold; baseline (speedup 1.0000x reference)
import jax
import jax.numpy as jnp
from jax.experimental import pallas as pl


def kernel(encoder_inputs, weight):
    raise NotImplementedError("write your pallas kernel here")


# speedup vs baseline: 9.6451x; 9.6451x over previous
import functools

import jax
import jax.numpy as jnp
from jax import lax
from jax.experimental import pallas as pl
from jax.experimental.pallas import tpu as pltpu
from jax.experimental.pallas import tpu_sc as plsc

_K = 8192
_D = 256
_N = 8192
_R = 256
_NTILES = _N // _R
_BETA = 0.25

_NC = 2
_NS = 16
_NW = _NC * _NS
_BPW = _N // _NW
_XCHUNK = 64


def _argmin_body(xsq_ref, wsq_ref, x_ref, w_ref, idx_ref, acc_ref):
    m = lax.dot_general(
        x_ref[...], w_ref[...],
        dimension_numbers=(((1,), (1,)), ((), ())),
        preferred_element_type=jnp.float32,
    )
    d = (xsq_ref[...] + wsq_ref[...]) - 2.0 * m
    minv = jnp.min(d, axis=1, keepdims=True)
    iota = lax.broadcasted_iota(jnp.int32, d.shape, 1)
    cand = jnp.where(d == minv, iota, _K)
    idx = jnp.min(cand, axis=1)
    idx_ref[0, 0, :] = idx

    @pl.when(pl.program_id(0) == 0)
    def _init():
        acc_ref[0, 0] = 0.0

    acc_ref[0, 0] += jnp.sum(minv)


def _distance_argmin(flat, xsq, wsq, weight):
    return pl.pallas_call(
        _argmin_body,
        grid=(_NTILES,),
        in_specs=[
            pl.BlockSpec((_R, 1), lambda i: (i, 0)),
            pl.BlockSpec((1, _K), lambda i: (0, 0)),
            pl.BlockSpec((_R, _D), lambda i: (i, 0)),
            pl.BlockSpec((_K, _D), lambda i: (0, 0)),
        ],
        out_specs=[
            pl.BlockSpec((1, 1, _R), lambda i: (i, 0, 0)),
            pl.BlockSpec(memory_space=pltpu.SMEM, block_shape=(1, 1),
                         index_map=lambda i: (0, 0)),
        ],
        out_shape=[
            jax.ShapeDtypeStruct((_NTILES, 1, _R), jnp.int32),
            jax.ShapeDtypeStruct((1, 1), jnp.float32),
        ],
    )(xsq, wsq, flat, weight)


@functools.cache
def _make_gather_st():
    mesh = plsc.VectorSubcoreMesh(core_axis_name="c", subcore_axis_name="s")

    @functools.partial(
        pl.kernel,
        mesh=mesh,
        out_type=jax.ShapeDtypeStruct((_N, _D), jnp.float32),
        scratch_types=[
            pltpu.VMEM((2, _BPW // 2), jnp.int32),
            pltpu.VMEM((_BPW, _D), jnp.float32),
            pltpu.VMEM((_XCHUNK, _D), jnp.float32),
            pltpu.SemaphoreType.DMA,
        ],
    )
    def _gather_st(w_hbm, idx_hbm, x_hbm, out_hbm, idx_v, rows_v, x_v, sem):
        wid = lax.axis_index("s") * _NC + lax.axis_index("c")
        base = wid * _BPW
        pltpu.sync_copy(idx_hbm.at[wid], idx_v)
        half = _BPW // 2
        c0 = pltpu.async_copy(w_hbm.at[idx_v.at[0]], rows_v.at[pl.ds(0, half)], sem)
        c1 = pltpu.async_copy(w_hbm.at[idx_v.at[1]], rows_v.at[pl.ds(half, half)], sem)
        c0.wait()
        c1.wait()
        for c in range(_BPW // _XCHUNK):
            pltpu.sync_copy(x_hbm.at[pl.ds(base + c * _XCHUNK, _XCHUNK)], x_v)

            def body(r, carry, c=c):
                row = c * _XCHUNK + r
                for j in range(_D // 16):
                    sl = pl.ds(j * 16, 16)
                    q = rows_v[row, sl]
                    xv = x_v[r, sl]
                    rows_v[row, sl] = xv + (q - xv)
                return carry

            lax.fori_loop(0, _XCHUNK, body, 0)
        pltpu.sync_copy(rows_v, out_hbm.at[pl.ds(base, _BPW)])

    return _gather_st


def kernel(encoder_inputs, weight):
    x = jnp.transpose(encoder_inputs, (0, 2, 3, 1))
    shape = x.shape
    flat = x.reshape(-1, _D)
    xsq = jnp.sum(flat ** 2, axis=1, keepdims=True)
    wsq = jnp.sum(weight ** 2, axis=1).reshape(1, _K)

    idx3, acc = _distance_argmin(flat, xsq, wsq, weight)
    idx = idx3.reshape(_NW, 2, _BPW // 2)

    out_flat = _make_gather_st()(weight, idx, flat)

    mean_d = acc[0, 0] / jnp.float32(_N * _D)
    loss = mean_d + _BETA * mean_d

    quantized = jnp.transpose(out_flat.reshape(shape), (0, 3, 1, 2))
    return (quantized, loss)
